# baseline (speedup 1.0000x reference)
import functools

import jax
import jax.numpy as jnp
from jax import lax
from jax.experimental import pallas as pl
from jax.experimental.pallas import tpu as pltpu, tpu_sc as plsc

_P = 2
_H = 480
_W = 640
_D = 128
_DY = int(_H / (_H + _W) * _D)
_DX = _D - _DY

_INFO = plsc.get_sparse_core_info()
_NC = _INFO.num_cores
_NS = _INFO.num_subcores
_NW = _NC * _NS
_LANES = 16

_VY = _P * _H + 1
_VX = _P * _W + 1
_NROWS = _VY + _VX
_TROWS = ((_NROWS + 32 + _NS * 8 - 1) // (_NS * 8)) * (_NS * 8)
_ROWS_PER_TILE = _TROWS // _NS

_CHUNK = 256
_GRP = 64


def _make_embed(n_tokens: int):
    tpw = n_tokens // _NW
    n_chunks = tpw // _CHUNK
    n_grp = _CHUNK // _GRP
    mesh = plsc.VectorSubcoreMesh(core_axis_name="c", subcore_axis_name="s")

    @functools.partial(
        pl.kernel,
        mesh=mesh,
        out_type=jax.ShapeDtypeStruct((n_tokens, _D), jnp.float32),
        compiler_params=pltpu.CompilerParams(use_tc_tiling_on_sc=False),
        scratch_types=[
            pltpu.VMEM((2, _CHUNK), jnp.int32),
            pltpu.VMEM((2, _CHUNK), jnp.int32),
            pltpu.VMEM((2, _CHUNK), jnp.int32),
            pltpu.VMEM((2, _CHUNK), jnp.int32),
            pltpu.VMEM((2, n_grp, _GRP), jnp.int32),
            pltpu.VMEM((2, n_grp, _GRP), jnp.int32),
            pltpu.VMEM((2, _CHUNK, _D), jnp.float32),
            pltpu.VMEM_SHARED((_TROWS, _D), jnp.float32),
        ] + [pltpu.SemaphoreType.DMA] * (2 + _CHUNK // _GRP + 1 + 2),
    )
    def embed(p_hbm, y_hbm, x_hbm, m_hbm, tab_hbm, out_hbm,
              pv, yv, xv, mv, iy, ix, obuf, tab, *sems):
        insems = sems[0:2]
        ysems = sems[2:2 + n_grp]
        xsem = sems[2 + n_grp]
        osems = sems[3 + n_grp:5 + n_grp]
        sid = lax.axis_index("s")
        wid = sid * _NC + lax.axis_index("c")

        lane = lax.iota(jnp.int32, _LANES)
        zrow = _NROWS + ((wid + lane) & 31)

        def start_inputs(t):
            b = t % 2
            base = wid * tpw + t * _CHUNK
            rows = pl.ds(base, _CHUNK)
            return [pltpu.async_copy(p_hbm.at[rows], pv.at[b], insems[b]),
                    pltpu.async_copy(y_hbm.at[rows], yv.at[b], insems[b]),
                    pltpu.async_copy(x_hbm.at[rows], xv.at[b], insems[b]),
                    pltpu.async_copy(m_hbm.at[rows], mv.at[b], insems[b])]

        def compute_indices(t):
            b = t % 2
            for j in range(n_grp):
                for k in range(_GRP // _LANES):
                    s0 = j * _GRP + k * _LANES
                    pp = pv[b, pl.ds(s0, _LANES)]
                    mm = mv[b, pl.ds(s0, _LANES)]
                    inv = (1 - mm) * zrow
                    iy[b, j, pl.ds(k * _LANES, _LANES)] = (
                        (pp * _H + yv[b, pl.ds(s0, _LANES)] + 1) * mm + inv)
                    ix[b, j, pl.ds(k * _LANES, _LANES)] = (
                        (pp * _W + xv[b, pl.ds(s0, _LANES)] + 1 + _VY) * mm
                        + inv)

        in_h = {0: start_inputs(0)}
        if n_chunks > 1:
            in_h[1] = start_inputs(1)
        stage = pl.ds(sid * _ROWS_PER_TILE, _ROWS_PER_TILE)
        pltpu.sync_copy(tab_hbm.at[stage], tab.at[stage])
        for h in in_h.pop(0):
            h.wait()
        compute_indices(0)
        plsc.subcore_barrier()

        out_h = {}
        for t in range(n_chunks):
            b = t % 2
            if t >= 2:
                for h in out_h.pop(t - 2):
                    h.wait()
            ycopies = []
            for j in range(n_grp):
                rows = pl.ds(j * _GRP, _GRP)
                ycopies.append(pltpu.async_copy(
                    tab.at[iy.at[b, j]], obuf.at[b, rows], ysems[j]))
            if t + 1 < n_chunks:
                for h in in_h.pop(t + 1):
                    h.wait()
                compute_indices(t + 1)
            if t + 2 < n_chunks:
                in_h[t + 2] = start_inputs(t + 2)
            base = wid * tpw + t * _CHUNK
            xcopies = []
            for j in range(n_grp):
                rows = pl.ds(j * _GRP, _GRP)
                ycopies[j].wait()
                xcopies.append(pltpu.async_copy(
                    tab.at[ix.at[b, j]], obuf.at[b, rows], xsem, add=True))
            wcopies = []
            for j in range(n_grp):
                rows = pl.ds(j * _GRP, _GRP)
                xcopies[j].wait()
                wcopies.append(pltpu.async_copy(
                    obuf.at[b, rows],
                    out_hbm.at[pl.ds(base + j * _GRP, _GRP)], osems[b]))
            out_h[t] = wcopies
        for t in (n_chunks - 2, n_chunks - 1):
            for h in out_h[t]:
                h.wait()

    return embed


def kernel(p, y, x, valid_mask, table_y, table_x):
    b, s = p.shape
    n = b * s
    m = valid_mask.reshape(n).astype(jnp.int32)
    tab = jnp.concatenate(
        [jnp.pad(table_y, ((0, 0), (0, _DX))),
         jnp.pad(table_x, ((0, _TROWS - _NROWS), (_DY, 0)))], axis=0)
    embed = _make_embed(n)
    out = embed(p.reshape(n), y.reshape(n), x.reshape(n), m, tab)
    return out.reshape(b, s, _D)


# speedup vs baseline: 1.0219x; 1.0219x over previous
import functools

import jax
import jax.numpy as jnp
from jax import lax
from jax.experimental import pallas as pl
from jax.experimental.pallas import tpu as pltpu, tpu_sc as plsc

_P = 2
_H = 480
_W = 640
_D = 128
_DY = int(_H / (_H + _W) * _D)
_DX = _D - _DY

_INFO = plsc.get_sparse_core_info()
_NC = _INFO.num_cores
_NS = _INFO.num_subcores
_NW = _NC * _NS
_LANES = 16

_VY = _P * _H + 1
_VX = _P * _W + 1
_NROWS = _VY + _VX
_TROWS = ((_NROWS + 32 + _NS * 8 - 1) // (_NS * 8)) * (_NS * 8)
_ROWS_PER_TILE = _TROWS // _NS

_CHUNK = 256
_GRP = 128


def _make_embed(n_tokens: int):
    tpw = n_tokens // _NW
    n_chunks = tpw // _CHUNK
    n_grp = _CHUNK // _GRP
    mesh = plsc.VectorSubcoreMesh(core_axis_name="c", subcore_axis_name="s")

    @functools.partial(
        pl.kernel,
        mesh=mesh,
        out_type=jax.ShapeDtypeStruct((n_tokens, _D), jnp.float32),
        compiler_params=pltpu.CompilerParams(use_tc_tiling_on_sc=False),
        scratch_types=[
            pltpu.VMEM((2, _CHUNK), jnp.int32),
            pltpu.VMEM((2, _CHUNK), jnp.int32),
            pltpu.VMEM((2, _CHUNK), jnp.int32),
            pltpu.VMEM((2, _CHUNK), jnp.int32),
            pltpu.VMEM((2, n_grp, _GRP), jnp.int32),
            pltpu.VMEM((2, n_grp, _GRP), jnp.int32),
            pltpu.VMEM((2, _CHUNK, _D), jnp.float32),
            pltpu.VMEM_SHARED((_TROWS, _D), jnp.float32),
        ] + [pltpu.SemaphoreType.DMA] * (2 + _CHUNK // _GRP + 1 + 2),
    )
    def embed(p_hbm, y_hbm, x_hbm, m_hbm, tab_hbm, out_hbm,
              pv, yv, xv, mv, iy, ix, obuf, tab, *sems):
        insems = sems[0:2]
        ysems = sems[2:2 + n_grp]
        xsem = sems[2 + n_grp]
        osems = sems[3 + n_grp:5 + n_grp]
        sid = lax.axis_index("s")
        wid = sid * _NC + lax.axis_index("c")

        lane = lax.iota(jnp.int32, _LANES)
        zrow = _NROWS + ((wid + lane) & 31)

        def start_inputs(t):
            b = t % 2
            base = wid * tpw + t * _CHUNK
            rows = pl.ds(base, _CHUNK)
            return [pltpu.async_copy(p_hbm.at[rows], pv.at[b], insems[b]),
                    pltpu.async_copy(y_hbm.at[rows], yv.at[b], insems[b]),
                    pltpu.async_copy(x_hbm.at[rows], xv.at[b], insems[b]),
                    pltpu.async_copy(m_hbm.at[rows], mv.at[b], insems[b])]

        def compute_indices(t):
            b = t % 2
            for j in range(n_grp):
                for k in range(_GRP // _LANES):
                    s0 = j * _GRP + k * _LANES
                    pp = pv[b, pl.ds(s0, _LANES)]
                    mm = mv[b, pl.ds(s0, _LANES)]
                    inv = (1 - mm) * zrow
                    iy[b, j, pl.ds(k * _LANES, _LANES)] = (
                        (pp * _H + yv[b, pl.ds(s0, _LANES)] + 1) * mm + inv)
                    ix[b, j, pl.ds(k * _LANES, _LANES)] = (
                        (pp * _W + xv[b, pl.ds(s0, _LANES)] + 1 + _VY) * mm
                        + inv)

        in_h = {0: start_inputs(0)}
        if n_chunks > 1:
            in_h[1] = start_inputs(1)
        stage = pl.ds(sid * _ROWS_PER_TILE, _ROWS_PER_TILE)
        pltpu.sync_copy(tab_hbm.at[stage], tab.at[stage])
        for h in in_h.pop(0):
            h.wait()
        compute_indices(0)
        plsc.subcore_barrier()

        out_h = {}
        for t in range(n_chunks):
            b = t % 2
            if t >= 2:
                out_h.pop(t - 2).wait()
            ycopies = []
            for j in range(n_grp):
                rows = pl.ds(j * _GRP, _GRP)
                ycopies.append(pltpu.async_copy(
                    tab.at[iy.at[b, j]], obuf.at[b, rows], ysems[j]))
            if t + 1 < n_chunks:
                for h in in_h.pop(t + 1):
                    h.wait()
                compute_indices(t + 1)
            if t + 2 < n_chunks:
                in_h[t + 2] = start_inputs(t + 2)
            xcopies = []
            for j in range(n_grp):
                rows = pl.ds(j * _GRP, _GRP)
                ycopies[j].wait()
                xcopies.append(pltpu.async_copy(
                    tab.at[ix.at[b, j]], obuf.at[b, rows], xsem, add=True))
            for c in xcopies:
                c.wait()
            base = wid * tpw + t * _CHUNK
            out_h[t] = pltpu.async_copy(
                obuf.at[b], out_hbm.at[pl.ds(base, _CHUNK)], osems[b])
        out_h[n_chunks - 2].wait()
        out_h[n_chunks - 1].wait()

    return embed


def kernel(p, y, x, valid_mask, table_y, table_x):
    b, s = p.shape
    n = b * s
    m = valid_mask.reshape(n).astype(jnp.int32)
    tab = jnp.concatenate(
        [jnp.pad(table_y, ((0, 0), (0, _DX))),
         jnp.pad(table_x, ((0, _TROWS - _NROWS), (_DY, 0)))], axis=0)
    embed = _make_embed(n)
    out = embed(p.reshape(n), y.reshape(n), x.reshape(n), m, tab)
    return out.reshape(b, s, _D)
